# Initial kernel scaffold; baseline (speedup 1.0000x reference)
import jax
import jax.numpy as jnp
from jax.experimental import pallas as pl


def kernel(x, pos, fc1_W, fc1_b, fc2_W, fc2_b, fd1_W, fd1_b, fd2_W, fd2_b, fg1_W, fg1_b, fg2_W, fg2_b, wq_W, wk_W, wv_W):
    raise NotImplementedError("write your pallas kernel here")


# speedup vs baseline: 15.4884x; 15.4884x over previous
import functools

import jax
import jax.numpy as jnp
import numpy as np
from jax import lax
from jax.experimental import pallas as pl
from jax.experimental.pallas import tpu as pltpu
from jax.experimental.pallas import tpu_sc as plsc

_B, _N, _D, _T, _K = 4, 2048, 64, 64, 16
_QA = 256
_QC = 256
_PW = 16
_W = _T + _PW
_WG = 128
_CHUNK = 128


def _knn_proj_body(posq_ref, poskt_ref, x_ref, fc1w_ref, fc1b_ref, wq_ref,
                   h_ref, q_ref, idx_ref):
    b = pl.program_id(0)
    posq = posq_ref[0]
    poskt = poskt_ref[0]
    s1 = ((posq[:, 0:1] * posq[:, 0:1] + posq[:, 1:2] * posq[:, 1:2])
          + posq[:, 2:3] * posq[:, 2:3])
    s2 = ((poskt[0:1, :] * poskt[0:1, :] + poskt[1:2, :] * poskt[1:2, :])
          + poskt[2:3, :] * poskt[2:3, :])
    qk = jnp.dot(posq, poskt, preferred_element_type=jnp.float32)
    d = s1 + s2 - 2.0 * qk

    iota = lax.broadcasted_iota(jnp.int32, (_QA, _N), 1)
    big = jnp.float32(np.inf)
    cols = []
    for _ in range(_K):
        m = jnp.min(d, axis=1, keepdims=True)
        eq = d == m
        idxj = jnp.min(jnp.where(eq, iota, _N), axis=1, keepdims=True)
        d = jnp.where(iota == idxj, big, d)
        cols.append(idxj)
    idx_ref[0] = jnp.concatenate(cols, axis=1) + b * _N

    h = (jnp.dot(x_ref[0], fc1w_ref[...], preferred_element_type=jnp.float32)
         + fc1b_ref[...])
    h_ref[0] = h
    q_ref[0] = jnp.dot(h, wq_ref[...], preferred_element_type=jnp.float32)


def _knn_proj(pos_pad, pos_kt, x, fc1w, fc1b, wq):
    grid = (_B, _N // _QA)
    return pl.pallas_call(
        _knn_proj_body,
        grid=grid,
        in_specs=[
            pl.BlockSpec((1, _QA, _PW), lambda b, i: (b, i, 0)),
            pl.BlockSpec((1, _PW, _N), lambda b, i: (b, 0, 0)),
            pl.BlockSpec((1, _QA, _D), lambda b, i: (b, i, 0)),
            pl.BlockSpec((_D, _T), lambda b, i: (0, 0)),
            pl.BlockSpec((1, _T), lambda b, i: (0, 0)),
            pl.BlockSpec((_T, _T), lambda b, i: (0, 0)),
        ],
        out_specs=[
            pl.BlockSpec((1, _QA, _T), lambda b, i: (b, i, 0)),
            pl.BlockSpec((1, _QA, _T), lambda b, i: (b, i, 0)),
            pl.BlockSpec((1, _QA, _K), lambda b, i: (b, i, 0)),
        ],
        out_shape=[
            jax.ShapeDtypeStruct((_B, _N, _T), jnp.float32),
            jax.ShapeDtypeStruct((_B, _N, _T), jnp.float32),
            jax.ShapeDtypeStruct((_B, _N, _K), jnp.int32),
        ],
    )(pos_pad, pos_kt, x, fc1w, fc1b, wq)


def _sc_gather(table, idx2d):
    rows = _B * _N * _K
    info = plsc.get_sparse_core_info()
    nw = info.num_cores * info.num_subcores
    per_w = rows // nw
    nch = per_w // _CHUNK
    mesh = plsc.VectorSubcoreMesh(core_axis_name="c", subcore_axis_name="s")

    @functools.partial(
        pl.kernel,
        mesh=mesh,
        out_type=jax.ShapeDtypeStruct((rows, _WG), jnp.float32),
        scratch_types=[
            pltpu.VMEM((nch, _CHUNK), jnp.int32),
            pltpu.VMEM((_CHUNK, _WG), jnp.float32),
            pltpu.SemaphoreType.DMA,
        ],
    )
    def gather_k(table_hbm, idx_hbm, out_hbm, idx_v, rows_v, sem):
        wid = lax.axis_index("s") * info.num_cores + lax.axis_index("c")
        base = wid * per_w
        pltpu.sync_copy(idx_hbm.at[pl.ds(wid * nch, nch)], idx_v)

        def body(c, carry):
            pltpu.async_copy(table_hbm.at[idx_v.at[c]], rows_v, sem).wait()
            pltpu.sync_copy(rows_v,
                            out_hbm.at[pl.ds(base + c * _CHUNK, _CHUNK)])
            return carry

        lax.fori_loop(0, nch, body, 0)

    return gather_k(table, idx2d)


def _attn_body(q_ref, g_ref, posq_ref, x_ref,
               wk_ref, wv_ref, fd1w_ref, fd1b_ref, fd2w_ref, fd2b_ref,
               fg1w_ref, fg1b_ref, fg2w_ref, fg2b_ref, fc2w_ref, fc2b_ref,
               res_ref, attn_ref):
    g = g_ref[0]
    hg = g[:, :_T]
    gpos = g[:, _T:_W]
    posq = posq_ref[0]

    rel = (jnp.reshape(posq, (_QC, 1, _PW))
           - jnp.reshape(gpos, (_QC, _K, _PW)))
    rel2 = jnp.reshape(rel, (_QC * _K, _PW))
    pe = (jnp.dot(jnp.maximum(
              jnp.dot(rel2, fd1w_ref[...],
                      preferred_element_type=jnp.float32) + fd1b_ref[...],
              0.0),
          fd2w_ref[...], preferred_element_type=jnp.float32)
          + fd2b_ref[...])

    kf = jnp.dot(hg, wk_ref[...], preferred_element_type=jnp.float32)
    vf = jnp.dot(hg, wv_ref[...], preferred_element_type=jnp.float32)
    q = q_ref[0]

    gin = (jnp.reshape(q, (_QC, 1, _T))
           - jnp.reshape(kf, (_QC, _K, _T))
           + jnp.reshape(pe, (_QC, _K, _T)))
    gin2 = jnp.reshape(gin, (_QC * _K, _T))
    logits = (jnp.dot(jnp.maximum(
                  jnp.dot(gin2, fg1w_ref[...],
                          preferred_element_type=jnp.float32) + fg1b_ref[...],
                  0.0),
              fg2w_ref[...], preferred_element_type=jnp.float32)
              + fg2b_ref[...])

    z = jnp.reshape(logits, (_QC, _K, _T)) * np.float32(1.0 / np.sqrt(_T))
    m = jnp.max(z, axis=1, keepdims=True)
    e = jnp.exp(z - m)
    attn = e / jnp.sum(e, axis=1, keepdims=True)
    attn_ref[0] = attn

    w = attn * (jnp.reshape(vf, (_QC, _K, _T)) + jnp.reshape(pe, (_QC, _K, _T)))
    r = jnp.sum(w, axis=1)
    res_ref[0] = (jnp.dot(r, fc2w_ref[...], preferred_element_type=jnp.float32)
                  + fc2b_ref[...] + x_ref[0])


def _attn(q, gath, pos_pad, x, wk, wv, fd1w, fd1b, fd2w, fd2b,
          fg1w, fg1b, fg2w, fg2b, fc2w, fc2b):
    grid = (_B, _N // _QC)
    wspec = lambda shape: pl.BlockSpec(shape, lambda b, i: tuple(0 for _ in shape))
    return pl.pallas_call(
        _attn_body,
        grid=grid,
        in_specs=[
            pl.BlockSpec((1, _QC, _T), lambda b, i: (b, i, 0)),
            pl.BlockSpec((1, _QC * _K, _WG), lambda b, i: (b, i, 0)),
            pl.BlockSpec((1, _QC, _PW), lambda b, i: (b, i, 0)),
            pl.BlockSpec((1, _QC, _D), lambda b, i: (b, i, 0)),
            wspec((_T, _T)), wspec((_T, _T)),
            wspec((_PW, _T)), wspec((1, _T)), wspec((_T, _T)), wspec((1, _T)),
            wspec((_T, _T)), wspec((1, _T)), wspec((_T, _T)), wspec((1, _T)),
            wspec((_T, _D)), wspec((1, _D)),
        ],
        out_specs=[
            pl.BlockSpec((1, _QC, _D), lambda b, i: (b, i, 0)),
            pl.BlockSpec((1, _QC, _K, _T), lambda b, i: (b, i, 0, 0)),
        ],
        out_shape=[
            jax.ShapeDtypeStruct((_B, _N, _D), jnp.float32),
            jax.ShapeDtypeStruct((_B, _N, _K, _T), jnp.float32),
        ],
    )(q, gath, pos_pad, x, wk, wv, fd1w, fd1b, fd2w, fd2b,
      fg1w, fg1b, fg2w, fg2b, fc2w, fc2b)


def kernel(x, pos, fc1_W, fc1_b, fc2_W, fc2_b, fd1_W, fd1_b, fd2_W, fd2_b,
           fg1_W, fg1_b, fg2_W, fg2_b, wq_W, wk_W, wv_W):
    pos_pad = jnp.pad(pos, ((0, 0), (0, 0), (0, _PW - 3)))
    pos_kt = jnp.transpose(pos_pad, (0, 2, 1))
    fd1p = jnp.pad(fd1_W, ((0, _PW - 3), (0, 0)))

    h, q, gidx = _knn_proj(pos_pad, pos_kt, x,
                           fc1_W, fc1_b.reshape(1, _T), wq_W)

    table = jnp.concatenate(
        [h, pos_pad, jnp.zeros((_B, _N, _WG - _W), jnp.float32)],
        axis=2).reshape(_B * _N, _WG)
    idx2d = gidx.reshape(_B * _N * _K // _CHUNK, _CHUNK)
    gath = _sc_gather(table, idx2d).reshape(_B, _N * _K, _WG)

    res, attn = _attn(q, gath, pos_pad, x, wk_W, wv_W,
                      fd1p, fd1_b.reshape(1, _T), fd2_W, fd2_b.reshape(1, _T),
                      fg1_W, fg1_b.reshape(1, _T), fg2_W, fg2_b.reshape(1, _T),
                      fc2_W, fc2_b.reshape(1, _D))
    return (res, attn)


# speedup vs baseline: 18.0104x; 1.1628x over previous
import functools

import jax
import jax.numpy as jnp
import numpy as np
from jax import lax
from jax.experimental import pallas as pl
from jax.experimental.pallas import tpu as pltpu
from jax.experimental.pallas import tpu_sc as plsc

_B, _N, _D, _T, _K = 4, 2048, 64, 64, 16
_QA = 256
_QC = 256
_PW = 16
_W = _T + _PW
_WG = 128
_CHUNK = 128


def _knn_proj_body(posq_ref, poskt_ref, x_ref, fc1w_ref, fc1b_ref, wq_ref,
                   table_ref, q_ref, idx_ref):
    b = pl.program_id(0)
    posq = posq_ref[0]
    poskt = poskt_ref[0]
    s1 = ((posq[:, 0:1] * posq[:, 0:1] + posq[:, 1:2] * posq[:, 1:2])
          + posq[:, 2:3] * posq[:, 2:3])
    s2 = ((poskt[0:1, :] * poskt[0:1, :] + poskt[1:2, :] * poskt[1:2, :])
          + poskt[2:3, :] * poskt[2:3, :])
    qk = jnp.dot(posq, poskt, preferred_element_type=jnp.float32)
    d = s1 + s2 - 2.0 * qk

    iota_f = lax.broadcasted_iota(jnp.int32, (_QA, _N), 1).astype(jnp.float32)
    big = jnp.float32(np.inf)
    big_i = jnp.float32(1e9)
    cols = []
    for _ in range(_K):
        m = jnp.min(d, axis=1, keepdims=True)
        c = jnp.where(d == m, iota_f, big_i)
        idxf = jnp.min(c, axis=1, keepdims=True)
        d = jnp.where(c == idxf, big, d)
        cols.append(idxf)
    idx = jnp.concatenate(cols, axis=1).astype(jnp.int32)
    idx_ref[0] = idx + b * _N

    h = (jnp.dot(x_ref[0], fc1w_ref[...], preferred_element_type=jnp.float32)
         + fc1b_ref[...])
    table_ref[0] = jnp.concatenate(
        [h, posq, jnp.zeros((_QA, _WG - _W), jnp.float32)], axis=1)
    q_ref[0] = jnp.dot(h, wq_ref[...], preferred_element_type=jnp.float32)


def _knn_proj(pos_pad, pos_kt, x, fc1w, fc1b, wq):
    grid = (_B, _N // _QA)
    return pl.pallas_call(
        _knn_proj_body,
        grid=grid,
        in_specs=[
            pl.BlockSpec((1, _QA, _PW), lambda b, i: (b, i, 0)),
            pl.BlockSpec((1, _PW, _N), lambda b, i: (b, 0, 0)),
            pl.BlockSpec((1, _QA, _D), lambda b, i: (b, i, 0)),
            pl.BlockSpec((_D, _T), lambda b, i: (0, 0)),
            pl.BlockSpec((1, _T), lambda b, i: (0, 0)),
            pl.BlockSpec((_T, _T), lambda b, i: (0, 0)),
        ],
        out_specs=[
            pl.BlockSpec((1, _QA, _WG), lambda b, i: (b, i, 0)),
            pl.BlockSpec((1, _QA, _T), lambda b, i: (b, i, 0)),
            pl.BlockSpec((1, _QA, _K), lambda b, i: (b, i, 0)),
        ],
        out_shape=[
            jax.ShapeDtypeStruct((_B, _N, _WG), jnp.float32),
            jax.ShapeDtypeStruct((_B, _N, _T), jnp.float32),
            jax.ShapeDtypeStruct((_B, _N, _K), jnp.int32),
        ],
    )(pos_pad, pos_kt, x, fc1w, fc1b, wq)


def _sc_gather(table, idx2d):
    rows = _B * _N * _K
    info = plsc.get_sparse_core_info()
    nw = info.num_cores * info.num_subcores
    per_w = rows // nw
    nch = per_w // _CHUNK
    mesh = plsc.VectorSubcoreMesh(core_axis_name="c", subcore_axis_name="s")

    @functools.partial(
        pl.kernel,
        mesh=mesh,
        out_type=jax.ShapeDtypeStruct((rows, _WG), jnp.float32),
        scratch_types=[
            pltpu.VMEM((nch, _CHUNK), jnp.int32),
            pltpu.VMEM((_CHUNK, _WG), jnp.float32),
            pltpu.VMEM((_CHUNK, _WG), jnp.float32),
            pltpu.SemaphoreType.DMA,
            pltpu.SemaphoreType.DMA,
        ],
    )
    def gather_k(table_hbm, idx_hbm, out_hbm, idx_v, rows0, rows1,
                 sem0, sem1):
        wid = lax.axis_index("s") * info.num_cores + lax.axis_index("c")
        base = wid * per_w
        pltpu.sync_copy(idx_hbm.at[pl.ds(wid * nch, nch)], idx_v)

        pltpu.async_copy(table_hbm.at[idx_v.at[0]], rows0, sem0)

        def body(g, carry):
            pltpu.make_async_copy(table_hbm.at[idx_v.at[g]], rows0,
                                  sem0).wait()
            pltpu.async_copy(table_hbm.at[idx_v.at[g + 1]], rows1, sem1)
            pltpu.sync_copy(rows0,
                            out_hbm.at[pl.ds(base + g * _CHUNK, _CHUNK)])
            pltpu.make_async_copy(table_hbm.at[idx_v.at[g + 1]], rows1,
                                  sem1).wait()

            @pl.when(g + 2 < nch)
            def _():
                pltpu.async_copy(table_hbm.at[idx_v.at[g + 2]], rows0, sem0)

            pltpu.sync_copy(
                rows1, out_hbm.at[pl.ds(base + (g + 1) * _CHUNK, _CHUNK)])
            return carry

        lax.fori_loop(0, nch // 2, lambda i, c: body(i * 2, c), 0)

    return gather_k(table, idx2d)


def _attn_body(q_ref, g_ref, posq_ref, x_ref,
               wk_ref, wv_ref, fd1w_ref, fd1b_ref, fd2w_ref, fd2b_ref,
               fg1w_ref, fg1b_ref, fg2w_ref, fg2b_ref, fc2w_ref, fc2b_ref,
               res_ref, attn_ref):
    g = g_ref[0]
    hg = g[:, :_T]
    gpos = g[:, _T:_W]
    posq = posq_ref[0]

    rel = (jnp.reshape(posq, (_QC, 1, _PW))
           - jnp.reshape(gpos, (_QC, _K, _PW)))
    rel2 = jnp.reshape(rel, (_QC * _K, _PW))
    pe = (jnp.dot(jnp.maximum(
              jnp.dot(rel2, fd1w_ref[...],
                      preferred_element_type=jnp.float32) + fd1b_ref[...],
              0.0),
          fd2w_ref[...], preferred_element_type=jnp.float32)
          + fd2b_ref[...])

    kf = jnp.dot(hg, wk_ref[...], preferred_element_type=jnp.float32)
    vf = jnp.dot(hg, wv_ref[...], preferred_element_type=jnp.float32)
    q = q_ref[0]

    gin = (jnp.reshape(q, (_QC, 1, _T))
           - jnp.reshape(kf, (_QC, _K, _T))
           + jnp.reshape(pe, (_QC, _K, _T)))
    gin2 = jnp.reshape(gin, (_QC * _K, _T))
    logits = (jnp.dot(jnp.maximum(
                  jnp.dot(gin2, fg1w_ref[...],
                          preferred_element_type=jnp.float32) + fg1b_ref[...],
                  0.0),
              fg2w_ref[...], preferred_element_type=jnp.float32)
              + fg2b_ref[...])

    z = jnp.reshape(logits, (_QC, _K, _T)) * np.float32(1.0 / np.sqrt(_T))
    m = jnp.max(z, axis=1, keepdims=True)
    e = jnp.exp(z - m)
    attn = e / jnp.sum(e, axis=1, keepdims=True)
    attn_ref[0] = attn

    w = attn * (jnp.reshape(vf, (_QC, _K, _T)) + jnp.reshape(pe, (_QC, _K, _T)))
    r = jnp.sum(w, axis=1)
    res_ref[0] = (jnp.dot(r, fc2w_ref[...], preferred_element_type=jnp.float32)
                  + fc2b_ref[...] + x_ref[0])


def _attn(q, gath, pos_pad, x, wk, wv, fd1w, fd1b, fd2w, fd2b,
          fg1w, fg1b, fg2w, fg2b, fc2w, fc2b):
    grid = (_B, _N // _QC)
    wspec = lambda shape: pl.BlockSpec(shape, lambda b, i: tuple(0 for _ in shape))
    return pl.pallas_call(
        _attn_body,
        grid=grid,
        in_specs=[
            pl.BlockSpec((1, _QC, _T), lambda b, i: (b, i, 0)),
            pl.BlockSpec((1, _QC * _K, _WG), lambda b, i: (b, i, 0)),
            pl.BlockSpec((1, _QC, _PW), lambda b, i: (b, i, 0)),
            pl.BlockSpec((1, _QC, _D), lambda b, i: (b, i, 0)),
            wspec((_T, _T)), wspec((_T, _T)),
            wspec((_PW, _T)), wspec((1, _T)), wspec((_T, _T)), wspec((1, _T)),
            wspec((_T, _T)), wspec((1, _T)), wspec((_T, _T)), wspec((1, _T)),
            wspec((_T, _D)), wspec((1, _D)),
        ],
        out_specs=[
            pl.BlockSpec((1, _QC, _D), lambda b, i: (b, i, 0)),
            pl.BlockSpec((1, _QC, _K, _T), lambda b, i: (b, i, 0, 0)),
        ],
        out_shape=[
            jax.ShapeDtypeStruct((_B, _N, _D), jnp.float32),
            jax.ShapeDtypeStruct((_B, _N, _K, _T), jnp.float32),
        ],
    )(q, gath, pos_pad, x, wk, wv, fd1w, fd1b, fd2w, fd2b,
      fg1w, fg1b, fg2w, fg2b, fc2w, fc2b)


def kernel(x, pos, fc1_W, fc1_b, fc2_W, fc2_b, fd1_W, fd1_b, fd2_W, fd2_b,
           fg1_W, fg1_b, fg2_W, fg2_b, wq_W, wk_W, wv_W):
    pos_pad = jnp.pad(pos, ((0, 0), (0, 0), (0, _PW - 3)))
    pos_kt = jnp.transpose(pos_pad, (0, 2, 1))
    fd1p = jnp.pad(fd1_W, ((0, _PW - 3), (0, 0)))

    table, q, gidx = _knn_proj(pos_pad, pos_kt, x,
                               fc1_W, fc1_b.reshape(1, _T), wq_W)

    table = table.reshape(_B * _N, _WG)
    idx2d = gidx.reshape(_B * _N * _K // _CHUNK, _CHUNK)
    gath = _sc_gather(table, idx2d).reshape(_B, _N * _K, _WG)

    res, attn = _attn(q, gath, pos_pad, x, wk_W, wv_W,
                      fd1p, fd1_b.reshape(1, _T), fd2_W, fd2_b.reshape(1, _T),
                      fg1_W, fg1_b.reshape(1, _T), fg2_W, fg2_b.reshape(1, _T),
                      fc2_W, fc2_b.reshape(1, _D))
    return (res, attn)


# speedup vs baseline: 18.4687x; 1.0254x over previous
import functools

import jax
import jax.numpy as jnp
import numpy as np
from jax import lax
from jax.experimental import pallas as pl
from jax.experimental.pallas import tpu as pltpu
from jax.experimental.pallas import tpu_sc as plsc

_B, _N, _D, _T, _K = 4, 2048, 64, 64, 16
_QA = 256
_QC = 256
_PW = 16
_WG = 128
_CHUNK = 128


def _knn_proj_body(posq_ref, poskt_ref, x_ref, fc1w_ref, fc1b_ref, wq_ref,
                   fd1w_ref, fd1b_ref, table_ref, qp_ref, idx_ref):
    b = pl.program_id(0)
    posq = posq_ref[0]
    poskt = poskt_ref[0]
    s1 = ((posq[:, 0:1] * posq[:, 0:1] + posq[:, 1:2] * posq[:, 1:2])
          + posq[:, 2:3] * posq[:, 2:3])
    s2 = ((poskt[0:1, :] * poskt[0:1, :] + poskt[1:2, :] * poskt[1:2, :])
          + poskt[2:3, :] * poskt[2:3, :])
    qk = jnp.dot(posq, poskt, preferred_element_type=jnp.float32)
    d = s1 + s2 - 2.0 * qk

    iota_f = lax.broadcasted_iota(jnp.int32, (_QA, _N), 1).astype(jnp.float32)
    big = jnp.float32(np.inf)
    big_i = jnp.float32(1e9)
    cols = []
    for _ in range(_K):
        m = jnp.min(d, axis=1, keepdims=True)
        c = jnp.where(d == m, iota_f, big_i)
        idxf = jnp.min(c, axis=1, keepdims=True)
        d = jnp.where(c == idxf, big, d)
        cols.append(idxf)
    idx = jnp.concatenate(cols, axis=1).astype(jnp.int32)
    idx_ref[0] = idx + b * _N

    h = (jnp.dot(x_ref[0], fc1w_ref[...], preferred_element_type=jnp.float32)
         + fc1b_ref[...])
    p1 = jnp.dot(posq, fd1w_ref[...], preferred_element_type=jnp.float32)
    table_ref[0] = jnp.concatenate([h, p1], axis=1)
    q = jnp.dot(h, wq_ref[...], preferred_element_type=jnp.float32)
    qp_ref[0] = jnp.concatenate([q, p1 + fd1b_ref[...]], axis=1)


def _knn_proj(pos_pad, pos_kt, x, fc1w, fc1b, wq, fd1w, fd1b):
    grid = (_B, _N // _QA)
    return pl.pallas_call(
        _knn_proj_body,
        grid=grid,
        in_specs=[
            pl.BlockSpec((1, _QA, _PW), lambda b, i: (b, i, 0)),
            pl.BlockSpec((1, _PW, _N), lambda b, i: (b, 0, 0)),
            pl.BlockSpec((1, _QA, _D), lambda b, i: (b, i, 0)),
            pl.BlockSpec((_D, _T), lambda b, i: (0, 0)),
            pl.BlockSpec((1, _T), lambda b, i: (0, 0)),
            pl.BlockSpec((_T, _T), lambda b, i: (0, 0)),
            pl.BlockSpec((_PW, _T), lambda b, i: (0, 0)),
            pl.BlockSpec((1, _T), lambda b, i: (0, 0)),
        ],
        out_specs=[
            pl.BlockSpec((1, _QA, _WG), lambda b, i: (b, i, 0)),
            pl.BlockSpec((1, _QA, _WG), lambda b, i: (b, i, 0)),
            pl.BlockSpec((1, _QA, _K), lambda b, i: (b, i, 0)),
        ],
        out_shape=[
            jax.ShapeDtypeStruct((_B, _N, _WG), jnp.float32),
            jax.ShapeDtypeStruct((_B, _N, _WG), jnp.float32),
            jax.ShapeDtypeStruct((_B, _N, _K), jnp.int32),
        ],
    )(pos_pad, pos_kt, x, fc1w, fc1b, wq, fd1w, fd1b)


def _sc_gather(table, idx2d):
    rows = idx2d.shape[0] * _CHUNK
    info = plsc.get_sparse_core_info()
    nw = info.num_cores * info.num_subcores
    per_w = rows // nw
    nch = per_w // _CHUNK
    mesh = plsc.VectorSubcoreMesh(core_axis_name="c", subcore_axis_name="s")

    @functools.partial(
        pl.kernel,
        mesh=mesh,
        out_type=jax.ShapeDtypeStruct((rows, _WG), jnp.float32),
        scratch_types=[
            pltpu.VMEM((nch, _CHUNK), jnp.int32),
            pltpu.VMEM((_CHUNK, _WG), jnp.float32),
            pltpu.VMEM((_CHUNK, _WG), jnp.float32),
            pltpu.SemaphoreType.DMA,
            pltpu.SemaphoreType.DMA,
        ],
    )
    def gather_k(table_hbm, idx_hbm, out_hbm, idx_v, rows0, rows1,
                 sem0, sem1):
        wid = lax.axis_index("s") * info.num_cores + lax.axis_index("c")
        base = wid * per_w
        pltpu.sync_copy(idx_hbm.at[pl.ds(wid * nch, nch)], idx_v)

        pltpu.async_copy(table_hbm.at[idx_v.at[0]], rows0, sem0)

        def body(g, carry):
            pltpu.make_async_copy(table_hbm.at[idx_v.at[g]], rows0,
                                  sem0).wait()
            pltpu.async_copy(table_hbm.at[idx_v.at[g + 1]], rows1, sem1)
            pltpu.sync_copy(rows0,
                            out_hbm.at[pl.ds(base + g * _CHUNK, _CHUNK)])
            pltpu.make_async_copy(table_hbm.at[idx_v.at[g + 1]], rows1,
                                  sem1).wait()

            @pl.when(g + 2 < nch)
            def _():
                pltpu.async_copy(table_hbm.at[idx_v.at[g + 2]], rows0, sem0)

            pltpu.sync_copy(
                rows1, out_hbm.at[pl.ds(base + (g + 1) * _CHUNK, _CHUNK)])
            return carry

        lax.fori_loop(0, nch // 2, lambda i, c: body(i * 2, c), 0)

    return gather_k(table, idx2d)


def _attn_body(qp_ref, g_ref, x_ref,
               wkv_ref, fd2w_ref, fd2b_ref,
               fg1w_ref, fg1b_ref, fg2w_ref, fg2b_ref, fc2w_ref, fc2b_ref,
               res_ref, attn_ref):
    g = g_ref[...]
    hg = g[:, :_T]
    p1g = g[:, _T:]
    qp = qp_ref[0]
    q = qp[:, :_T]
    p1qb = qp[:, _T:]

    t1 = (jnp.reshape(p1qb, (_QC, 1, _T))
          - jnp.reshape(p1g, (_QC, _K, _T)))
    t1 = jnp.maximum(jnp.reshape(t1, (_QC * _K, _T)), 0.0)
    pe = (jnp.dot(t1, fd2w_ref[...], preferred_element_type=jnp.float32)
          + fd2b_ref[...])

    kv = jnp.dot(hg, wkv_ref[...], preferred_element_type=jnp.float32)
    kf = kv[:, :_T]
    vf = kv[:, _T:]

    gin = (jnp.reshape(q, (_QC, 1, _T))
           - jnp.reshape(kf, (_QC, _K, _T))
           + jnp.reshape(pe, (_QC, _K, _T)))
    gin2 = jnp.reshape(gin, (_QC * _K, _T))
    logits = (jnp.dot(jnp.maximum(
                  jnp.dot(gin2, fg1w_ref[...],
                          preferred_element_type=jnp.float32) + fg1b_ref[...],
                  0.0),
              fg2w_ref[...], preferred_element_type=jnp.float32)
              + fg2b_ref[...])

    z = jnp.reshape(logits, (_QC, _K, _T)) * np.float32(1.0 / np.sqrt(_T))
    m = jnp.max(z, axis=1, keepdims=True)
    e = jnp.exp(z - m)
    attn = e / jnp.sum(e, axis=1, keepdims=True)
    attn_ref[0] = attn

    w = attn * (jnp.reshape(vf, (_QC, _K, _T)) + jnp.reshape(pe, (_QC, _K, _T)))
    r = jnp.sum(w, axis=1)
    res_ref[0] = (jnp.dot(r, fc2w_ref[...], preferred_element_type=jnp.float32)
                  + fc2b_ref[...] + x_ref[0])


def _attn(qp, gath, x, wkv, fd2w, fd2b, fg1w, fg1b, fg2w, fg2b, fc2w, fc2b):
    grid = (_B, _N // _QC)
    nblk = _N // _QC
    wspec = lambda shape: pl.BlockSpec(shape, lambda b, i: tuple(0 for _ in shape))
    return pl.pallas_call(
        _attn_body,
        grid=grid,
        in_specs=[
            pl.BlockSpec((1, _QC, _WG), lambda b, i: (b, i, 0)),
            pl.BlockSpec((_QC * _K, _WG), lambda b, i: (b * nblk + i, 0)),
            pl.BlockSpec((1, _QC, _D), lambda b, i: (b, i, 0)),
            wspec((_T, 2 * _T)),
            wspec((_T, _T)), wspec((1, _T)),
            wspec((_T, _T)), wspec((1, _T)), wspec((_T, _T)), wspec((1, _T)),
            wspec((_T, _D)), wspec((1, _D)),
        ],
        out_specs=[
            pl.BlockSpec((1, _QC, _D), lambda b, i: (b, i, 0)),
            pl.BlockSpec((1, _QC, _K, _T), lambda b, i: (b, i, 0, 0)),
        ],
        out_shape=[
            jax.ShapeDtypeStruct((_B, _N, _D), jnp.float32),
            jax.ShapeDtypeStruct((_B, _N, _K, _T), jnp.float32),
        ],
    )(qp, gath, x, wkv, fd2w, fd2b, fg1w, fg1b, fg2w, fg2b, fc2w, fc2b)


def kernel(x, pos, fc1_W, fc1_b, fc2_W, fc2_b, fd1_W, fd1_b, fd2_W, fd2_b,
           fg1_W, fg1_b, fg2_W, fg2_b, wq_W, wk_W, wv_W):
    pos_pad = jnp.pad(pos, ((0, 0), (0, 0), (0, _PW - 3)))
    pos_kt = jnp.transpose(pos_pad, (0, 2, 1))
    fd1p = jnp.pad(fd1_W, ((0, _PW - 3), (0, 0)))

    table, qp, gidx = _knn_proj(pos_pad, pos_kt, x,
                                fc1_W, fc1_b.reshape(1, _T), wq_W,
                                fd1p, fd1_b.reshape(1, _T))

    table = table.reshape(_B * _N, _WG)
    idx2d = gidx.reshape(_B * _N * _K // _CHUNK, _CHUNK)
    gath = _sc_gather(table, idx2d)

    res, attn = _attn(qp, gath, x, jnp.concatenate([wk_W, wv_W], axis=1),
                      fd2_W, fd2_b.reshape(1, _T),
                      fg1_W, fg1_b.reshape(1, _T), fg2_W, fg2_b.reshape(1, _T),
                      fc2_W, fc2_b.reshape(1, _D))
    return (res, attn)


# speedup vs baseline: 18.5904x; 1.0066x over previous
import functools

import jax
import jax.numpy as jnp
import numpy as np
from jax import lax
from jax.experimental import pallas as pl
from jax.experimental.pallas import tpu as pltpu
from jax.experimental.pallas import tpu_sc as plsc

_B, _N, _D, _T, _K = 4, 2048, 64, 64, 16
_QA = 256
_QC = 256
_PW = 16
_WG = 128
_CHUNK = 128


def _knn_proj_body(posq_ref, poskt_ref, x_ref, fc1w_ref, fc1b_ref, wq_ref,
                   fd1w_ref, fd1b_ref, table_ref, qp_ref, idx_ref):
    b = pl.program_id(0)
    posq = posq_ref[0]
    poskt = poskt_ref[0]
    s1 = ((posq[:, 0:1] * posq[:, 0:1] + posq[:, 1:2] * posq[:, 1:2])
          + posq[:, 2:3] * posq[:, 2:3])
    s2 = ((poskt[0:1, :] * poskt[0:1, :] + poskt[1:2, :] * poskt[1:2, :])
          + poskt[2:3, :] * poskt[2:3, :])
    qk = jnp.dot(posq, poskt, preferred_element_type=jnp.float32)
    d = s1 + s2 - 2.0 * qk

    iota_f = lax.broadcasted_iota(jnp.int32, (_QA, _N), 1).astype(jnp.float32)
    big = jnp.float32(np.inf)
    big_i = jnp.float32(1e9)
    cols = []
    for _ in range(_K):
        m = jnp.min(d, axis=1, keepdims=True)
        c = jnp.where(d == m, iota_f, big_i)
        idxf = jnp.min(c, axis=1, keepdims=True)
        d = jnp.where(c == idxf, big, d)
        cols.append(idxf)
    idx = jnp.concatenate(cols, axis=1).astype(jnp.int32)
    idx_ref[0] = idx + b * _N

    h = (jnp.dot(x_ref[0], fc1w_ref[...], preferred_element_type=jnp.float32)
         + fc1b_ref[...])
    p1 = jnp.dot(posq, fd1w_ref[...], preferred_element_type=jnp.float32)
    table_ref[0] = jnp.concatenate([h, p1], axis=1)
    q = jnp.dot(h, wq_ref[...], preferred_element_type=jnp.float32)
    qp_ref[0] = jnp.concatenate([q, p1 + fd1b_ref[...]], axis=1)


def _knn_proj(pos_pad, pos_kt, x, fc1w, fc1b, wq, fd1w, fd1b):
    grid = (_B, _N // _QA)
    return pl.pallas_call(
        _knn_proj_body,
        grid=grid,
        in_specs=[
            pl.BlockSpec((1, _QA, _PW), lambda b, i: (b, i, 0)),
            pl.BlockSpec((1, _PW, _N), lambda b, i: (b, 0, 0)),
            pl.BlockSpec((1, _QA, _D), lambda b, i: (b, i, 0)),
            pl.BlockSpec((_D, _T), lambda b, i: (0, 0)),
            pl.BlockSpec((1, _T), lambda b, i: (0, 0)),
            pl.BlockSpec((_T, _T), lambda b, i: (0, 0)),
            pl.BlockSpec((_PW, _T), lambda b, i: (0, 0)),
            pl.BlockSpec((1, _T), lambda b, i: (0, 0)),
        ],
        out_specs=[
            pl.BlockSpec((1, _QA, _WG), lambda b, i: (b, i, 0)),
            pl.BlockSpec((1, _QA, _WG), lambda b, i: (b, i, 0)),
            pl.BlockSpec((1, _QA, _K), lambda b, i: (b, i, 0)),
        ],
        out_shape=[
            jax.ShapeDtypeStruct((_B, _N, _WG), jnp.float32),
            jax.ShapeDtypeStruct((_B, _N, _WG), jnp.float32),
            jax.ShapeDtypeStruct((_B, _N, _K), jnp.int32),
        ],
    )(pos_pad, pos_kt, x, fc1w, fc1b, wq, fd1w, fd1b)


def _sc_gather(table, idx2d):
    rows = idx2d.shape[0] * _CHUNK
    info = plsc.get_sparse_core_info()
    nw = info.num_cores * info.num_subcores
    per_w = rows // nw
    nch = per_w // _CHUNK
    mesh = plsc.VectorSubcoreMesh(core_axis_name="c", subcore_axis_name="s")

    @functools.partial(
        pl.kernel,
        mesh=mesh,
        out_type=jax.ShapeDtypeStruct((rows, _WG), jnp.float32),
        scratch_types=[
            pltpu.VMEM((nch, _CHUNK), jnp.int32),
            pltpu.VMEM((_CHUNK, _WG), jnp.float32),
            pltpu.VMEM((_CHUNK, _WG), jnp.float32),
            pltpu.SemaphoreType.DMA,
            pltpu.SemaphoreType.DMA,
        ],
    )
    def gather_k(table_hbm, idx_hbm, out_hbm, idx_v, rows0, rows1,
                 sem0, sem1):
        wid = lax.axis_index("s") * info.num_cores + lax.axis_index("c")
        base = wid * per_w
        pltpu.sync_copy(idx_hbm.at[pl.ds(wid * nch, nch)], idx_v)

        pltpu.async_copy(table_hbm.at[idx_v.at[0]], rows0, sem0)

        def body(g, carry):
            pltpu.make_async_copy(table_hbm.at[idx_v.at[g]], rows0,
                                  sem0).wait()
            pltpu.async_copy(table_hbm.at[idx_v.at[g + 1]], rows1, sem1)
            pltpu.sync_copy(rows0,
                            out_hbm.at[pl.ds(base + g * _CHUNK, _CHUNK)])
            pltpu.make_async_copy(table_hbm.at[idx_v.at[g + 1]], rows1,
                                  sem1).wait()

            @pl.when(g + 2 < nch)
            def _():
                pltpu.async_copy(table_hbm.at[idx_v.at[g + 2]], rows0, sem0)

            pltpu.sync_copy(
                rows1, out_hbm.at[pl.ds(base + (g + 1) * _CHUNK, _CHUNK)])
            return carry

        lax.fori_loop(0, nch // 2, lambda i, c: body(i * 2, c), 0)

    return gather_k(table, idx2d)


def _attn_body(qp_ref, g_ref, x_ref,
               wkv_ref, fd2w_ref, fd2b_ref,
               fg1w_ref, fg1b_ref, fg2w_ref, fg2b_ref, fc2w_ref, fc2b_ref,
               res_ref, attn_ref):
    g = g_ref[...]
    hg = g[:, :_T]
    p1g = g[:, _T:]
    qp = qp_ref[0]
    q = qp[:, :_T]
    p1qb = qp[:, _T:]

    t1 = (jnp.reshape(p1qb, (_QC, 1, _T))
          - jnp.reshape(p1g, (_QC, _K, _T)))
    t1 = jnp.maximum(jnp.reshape(t1, (_QC * _K, _T)), 0.0)
    pe = (jnp.dot(t1, fd2w_ref[...], preferred_element_type=jnp.float32)
          + fd2b_ref[...])

    kv = jnp.dot(hg, wkv_ref[...], preferred_element_type=jnp.float32)
    kf = kv[:, :_T]
    vf = kv[:, _T:]

    gin = (jnp.reshape(q, (_QC, 1, _T))
           - jnp.reshape(kf, (_QC, _K, _T))
           + jnp.reshape(pe, (_QC, _K, _T)))
    gin2 = jnp.reshape(gin, (_QC * _K, _T))
    logits = (jnp.dot(jnp.maximum(
                  jnp.dot(gin2, fg1w_ref[...],
                          preferred_element_type=jnp.float32) + fg1b_ref[...],
                  0.0),
              fg2w_ref[...], preferred_element_type=jnp.float32)
              + fg2b_ref[...])

    z = jnp.reshape(logits, (_QC, _K, _T)) * np.float32(1.0 / np.sqrt(_T))
    m = jnp.max(z, axis=1, keepdims=True)
    e = jnp.exp(z - m)
    attn = e / jnp.sum(e, axis=1, keepdims=True)
    for k in range(_K):
        attn_ref[0, k] = jnp.transpose(attn[:, k, :])

    w = attn * (jnp.reshape(vf, (_QC, _K, _T)) + jnp.reshape(pe, (_QC, _K, _T)))
    r = jnp.sum(w, axis=1)
    res_ref[0] = (jnp.dot(r, fc2w_ref[...], preferred_element_type=jnp.float32)
                  + fc2b_ref[...] + x_ref[0])


def _attn(qp, gath, x, wkv, fd2w, fd2b, fg1w, fg1b, fg2w, fg2b, fc2w, fc2b):
    grid = (_B, _N // _QC)
    nblk = _N // _QC
    wspec = lambda shape: pl.BlockSpec(shape, lambda b, i: tuple(0 for _ in shape))
    return pl.pallas_call(
        _attn_body,
        grid=grid,
        in_specs=[
            pl.BlockSpec((1, _QC, _WG), lambda b, i: (b, i, 0)),
            pl.BlockSpec((_QC * _K, _WG), lambda b, i: (b * nblk + i, 0)),
            pl.BlockSpec((1, _QC, _D), lambda b, i: (b, i, 0)),
            wspec((_T, 2 * _T)),
            wspec((_T, _T)), wspec((1, _T)),
            wspec((_T, _T)), wspec((1, _T)), wspec((_T, _T)), wspec((1, _T)),
            wspec((_T, _D)), wspec((1, _D)),
        ],
        out_specs=[
            pl.BlockSpec((1, _QC, _D), lambda b, i: (b, i, 0)),
            pl.BlockSpec((1, _K, _T, _QC), lambda b, i: (b, 0, 0, i)),
        ],
        out_shape=[
            jax.ShapeDtypeStruct((_B, _N, _D), jnp.float32),
            jax.ShapeDtypeStruct((_B, _K, _T, _N), jnp.float32),
        ],
    )(qp, gath, x, wkv, fd2w, fd2b, fg1w, fg1b, fg2w, fg2b, fc2w, fc2b)


def kernel(x, pos, fc1_W, fc1_b, fc2_W, fc2_b, fd1_W, fd1_b, fd2_W, fd2_b,
           fg1_W, fg1_b, fg2_W, fg2_b, wq_W, wk_W, wv_W):
    pos_pad = jnp.pad(pos, ((0, 0), (0, 0), (0, _PW - 3)))
    pos_kt = jnp.transpose(pos_pad, (0, 2, 1))
    fd1p = jnp.pad(fd1_W, ((0, _PW - 3), (0, 0)))

    table, qp, gidx = _knn_proj(pos_pad, pos_kt, x,
                                fc1_W, fc1_b.reshape(1, _T), wq_W,
                                fd1p, fd1_b.reshape(1, _T))

    table = table.reshape(_B * _N, _WG)
    idx2d = gidx.reshape(_B * _N * _K // _CHUNK, _CHUNK)
    gath = _sc_gather(table, idx2d)

    res, attn_t = _attn(qp, gath, x, jnp.concatenate([wk_W, wv_W], axis=1),
                        fd2_W, fd2_b.reshape(1, _T),
                        fg1_W, fg1_b.reshape(1, _T), fg2_W, fg2_b.reshape(1, _T),
                        fc2_W, fc2_b.reshape(1, _D))
    return (res, jnp.transpose(attn_t, (0, 3, 1, 2)))


# speedup vs baseline: 19.0185x; 1.0230x over previous
import functools

import jax
import jax.numpy as jnp
import numpy as np
from jax import lax
from jax.experimental import pallas as pl
from jax.experimental.pallas import tpu as pltpu
from jax.experimental.pallas import tpu_sc as plsc

_B, _N, _D, _T, _K = 4, 2048, 64, 64, 16
_QA = 256
_QC = 256
_PW = 16
_WG = 128
_CHUNK = 128


def _knn_proj_body(posq_ref, poskt_ref, x_ref, fc1w_ref, fc1b_ref,
                   fd1w_ref, table_ref, idx_ref):
    b = pl.program_id(0)
    posq = posq_ref[0]
    poskt = poskt_ref[0]
    s1 = ((posq[:, 0:1] * posq[:, 0:1] + posq[:, 1:2] * posq[:, 1:2])
          + posq[:, 2:3] * posq[:, 2:3])
    s2 = ((poskt[0:1, :] * poskt[0:1, :] + poskt[1:2, :] * poskt[1:2, :])
          + poskt[2:3, :] * poskt[2:3, :])
    qk = jnp.dot(posq, poskt, preferred_element_type=jnp.float32)
    d = s1 + s2 - 2.0 * qk

    iota_f = lax.broadcasted_iota(jnp.int32, (_QA, _N), 1).astype(jnp.float32)
    big = jnp.float32(np.inf)
    big_i = jnp.float32(1e9)
    cols = []
    for _ in range(_K):
        m = jnp.min(d, axis=1, keepdims=True)
        c = jnp.where(d == m, iota_f, big_i)
        idxf = jnp.min(c, axis=1, keepdims=True)
        d = jnp.where(c == idxf, big, d)
        cols.append(idxf)
    idx = jnp.concatenate(cols, axis=1).astype(jnp.int32)
    idx_ref[0] = idx + b * _N

    h = (jnp.dot(x_ref[0], fc1w_ref[...], preferred_element_type=jnp.float32)
         + fc1b_ref[...])
    p1 = jnp.dot(posq, fd1w_ref[...], preferred_element_type=jnp.float32)
    table_ref[0] = jnp.concatenate([h, p1], axis=1)


def _knn_proj(pos_pad, pos_kt, x, fc1w, fc1b, fd1w):
    grid = (_B, _N // _QA)
    return pl.pallas_call(
        _knn_proj_body,
        grid=grid,
        in_specs=[
            pl.BlockSpec((1, _QA, _PW), lambda b, i: (b, i, 0)),
            pl.BlockSpec((1, _PW, _N), lambda b, i: (b, 0, 0)),
            pl.BlockSpec((1, _QA, _D), lambda b, i: (b, i, 0)),
            pl.BlockSpec((_D, _T), lambda b, i: (0, 0)),
            pl.BlockSpec((1, _T), lambda b, i: (0, 0)),
            pl.BlockSpec((_PW, _T), lambda b, i: (0, 0)),
        ],
        out_specs=[
            pl.BlockSpec((1, _QA, _WG), lambda b, i: (b, i, 0)),
            pl.BlockSpec((1, _QA, _K), lambda b, i: (b, i, 0)),
        ],
        out_shape=[
            jax.ShapeDtypeStruct((_B, _N, _WG), jnp.float32),
            jax.ShapeDtypeStruct((_B, _N, _K), jnp.int32),
        ],
    )(pos_pad, pos_kt, x, fc1w, fc1b, fd1w)


def _sc_gather(table, idx2d):
    rows = idx2d.shape[0] * _CHUNK
    info = plsc.get_sparse_core_info()
    nw = info.num_cores * info.num_subcores
    per_w = rows // nw
    nch = per_w // _CHUNK
    mesh = plsc.VectorSubcoreMesh(core_axis_name="c", subcore_axis_name="s")

    @functools.partial(
        pl.kernel,
        mesh=mesh,
        out_type=jax.ShapeDtypeStruct((rows, _WG), jnp.float32),
        scratch_types=[
            pltpu.VMEM((nch, _CHUNK), jnp.int32),
            pltpu.VMEM((_CHUNK, _WG), jnp.float32),
            pltpu.VMEM((_CHUNK, _WG), jnp.float32),
            pltpu.SemaphoreType.DMA,
            pltpu.SemaphoreType.DMA,
        ],
    )
    def gather_k(table_hbm, idx_hbm, out_hbm, idx_v, rows0, rows1,
                 sem0, sem1):
        wid = lax.axis_index("s") * info.num_cores + lax.axis_index("c")
        base = wid * per_w
        pltpu.sync_copy(idx_hbm.at[pl.ds(wid * nch, nch)], idx_v)

        pltpu.async_copy(table_hbm.at[idx_v.at[0]], rows0, sem0)

        def body(g, carry):
            pltpu.make_async_copy(table_hbm.at[idx_v.at[g]], rows0,
                                  sem0).wait()
            pltpu.async_copy(table_hbm.at[idx_v.at[g + 1]], rows1, sem1)
            pltpu.sync_copy(rows0,
                            out_hbm.at[pl.ds(base + g * _CHUNK, _CHUNK)])
            pltpu.make_async_copy(table_hbm.at[idx_v.at[g + 1]], rows1,
                                  sem1).wait()

            @pl.when(g + 2 < nch)
            def _():
                pltpu.async_copy(table_hbm.at[idx_v.at[g + 2]], rows0, sem0)

            pltpu.sync_copy(
                rows1, out_hbm.at[pl.ds(base + (g + 1) * _CHUNK, _CHUNK)])
            return carry

        lax.fori_loop(0, nch // 2, lambda i, c: body(i * 2, c), 0)

    return gather_k(table, idx2d)


def _attn_body(tq_ref, g_ref, x_ref,
               wq_ref, fd1b_ref, wkv_ref, fd2w_ref, fd2b_ref,
               fg1w_ref, fg1b_ref, fg2w_ref, fg2b_ref, fc2w_ref, fc2b_ref,
               res_ref, attn_ref):
    g = g_ref[...]
    hg = g[:, :_T]
    p1g = g[:, _T:]
    tq = tq_ref[...]
    q = jnp.dot(tq[:, :_T], wq_ref[...], preferred_element_type=jnp.float32)
    p1qb = tq[:, _T:] + fd1b_ref[...]

    t1 = (jnp.reshape(p1qb, (_QC, 1, _T))
          - jnp.reshape(p1g, (_QC, _K, _T)))
    t1 = jnp.maximum(jnp.reshape(t1, (_QC * _K, _T)), 0.0)
    pe = (jnp.dot(t1, fd2w_ref[...], preferred_element_type=jnp.float32)
          + fd2b_ref[...])

    kv = jnp.dot(hg, wkv_ref[...], preferred_element_type=jnp.float32)
    kf = kv[:, :_T]
    vf = kv[:, _T:]

    gin = (jnp.reshape(q, (_QC, 1, _T))
           - jnp.reshape(kf, (_QC, _K, _T))
           + jnp.reshape(pe, (_QC, _K, _T)))
    gin2 = jnp.reshape(gin, (_QC * _K, _T))
    logits = (jnp.dot(jnp.maximum(
                  jnp.dot(gin2, fg1w_ref[...],
                          preferred_element_type=jnp.float32) + fg1b_ref[...],
                  0.0),
              fg2w_ref[...], preferred_element_type=jnp.float32)
              + fg2b_ref[...])

    z = jnp.reshape(logits, (_QC, _K, _T)) * np.float32(1.0 / np.sqrt(_T))
    m = jnp.max(z, axis=1, keepdims=True)
    e = jnp.exp(z - m)
    attn = e / jnp.sum(e, axis=1, keepdims=True)
    for k in range(_K):
        attn_ref[0, k] = jnp.transpose(attn[:, k, :])

    w = attn * (jnp.reshape(vf, (_QC, _K, _T)) + jnp.reshape(pe, (_QC, _K, _T)))
    r = jnp.sum(w, axis=1)
    res_ref[0] = (jnp.dot(r, fc2w_ref[...], preferred_element_type=jnp.float32)
                  + fc2b_ref[...] + x_ref[0])


def _attn(table, gath, x, wq, fd1b, wkv, fd2w, fd2b,
          fg1w, fg1b, fg2w, fg2b, fc2w, fc2b):
    grid = (_B, _N // _QC)
    nblk = _N // _QC
    wspec = lambda shape: pl.BlockSpec(shape, lambda b, i: tuple(0 for _ in shape))
    return pl.pallas_call(
        _attn_body,
        grid=grid,
        in_specs=[
            pl.BlockSpec((_QC, _WG), lambda b, i: (b * nblk + i, 0)),
            pl.BlockSpec((_QC * _K, _WG), lambda b, i: (b * nblk + i, 0)),
            pl.BlockSpec((1, _QC, _D), lambda b, i: (b, i, 0)),
            wspec((_T, _T)), wspec((1, _T)),
            wspec((_T, 2 * _T)),
            wspec((_T, _T)), wspec((1, _T)),
            wspec((_T, _T)), wspec((1, _T)), wspec((_T, _T)), wspec((1, _T)),
            wspec((_T, _D)), wspec((1, _D)),
        ],
        out_specs=[
            pl.BlockSpec((1, _QC, _D), lambda b, i: (b, i, 0)),
            pl.BlockSpec((1, _K, _T, _QC), lambda b, i: (b, 0, 0, i)),
        ],
        out_shape=[
            jax.ShapeDtypeStruct((_B, _N, _D), jnp.float32),
            jax.ShapeDtypeStruct((_B, _K, _T, _N), jnp.float32),
        ],
    )(table, gath, x, wq, fd1b, wkv, fd2w, fd2b,
      fg1w, fg1b, fg2w, fg2b, fc2w, fc2b)


def kernel(x, pos, fc1_W, fc1_b, fc2_W, fc2_b, fd1_W, fd1_b, fd2_W, fd2_b,
           fg1_W, fg1_b, fg2_W, fg2_b, wq_W, wk_W, wv_W):
    pos_pad = jnp.pad(pos, ((0, 0), (0, 0), (0, _PW - 3)))
    pos_kt = jnp.transpose(pos_pad, (0, 2, 1))
    fd1p = jnp.pad(fd1_W, ((0, _PW - 3), (0, 0)))

    table, gidx = _knn_proj(pos_pad, pos_kt, x,
                            fc1_W, fc1_b.reshape(1, _T), fd1p)

    table = table.reshape(_B * _N, _WG)
    idx2d = gidx.reshape(_B * _N * _K // _CHUNK, _CHUNK)
    gath = _sc_gather(table, idx2d)

    res, attn_t = _attn(table, gath, x, wq_W, fd1_b.reshape(1, _T),
                        jnp.concatenate([wk_W, wv_W], axis=1),
                        fd2_W, fd2_b.reshape(1, _T),
                        fg1_W, fg1_b.reshape(1, _T), fg2_W, fg2_b.reshape(1, _T),
                        fc2_W, fc2_b.reshape(1, _D))
    return (res, jnp.transpose(attn_t, (0, 3, 1, 2)))


# speedup vs baseline: 20.1581x; 1.0599x over previous
import functools

import jax
import jax.numpy as jnp
import numpy as np
from jax import lax
from jax.experimental import pallas as pl
from jax.experimental.pallas import tpu as pltpu
from jax.experimental.pallas import tpu_sc as plsc

_B, _N, _D, _T, _K = 4, 2048, 64, 64, 16
_QA = 256
_QC = 256
_PW = 16
_WG = 128
_CHUNK = 128


def _knn_proj_body(posq_ref, poskt_ref, x_ref, fc1w_ref, fc1b_ref,
                   fd1w_ref, table_ref, idx_ref):
    b = pl.program_id(0)
    posq = posq_ref[0]
    poskt = poskt_ref[0]
    s1 = ((posq[:, 0:1] * posq[:, 0:1] + posq[:, 1:2] * posq[:, 1:2])
          + posq[:, 2:3] * posq[:, 2:3])
    s2 = ((poskt[0:1, :] * poskt[0:1, :] + poskt[1:2, :] * poskt[1:2, :])
          + poskt[2:3, :] * poskt[2:3, :])
    qk = jnp.dot(posq, poskt, preferred_element_type=jnp.float32)
    d = s1 + s2 - 2.0 * qk

    big = jnp.float32(np.inf)
    big_i = jnp.float32(1e9)
    iota_f = lax.broadcasted_iota(jnp.int32, (_QA, _N), 1).astype(jnp.float32)
    lane_f = lax.broadcasted_iota(jnp.int32, (_QA, 128), 1).astype(jnp.float32)

    nlv = 5
    fv = [jnp.full((_QA, 128), big, jnp.float32) for _ in range(nlv)]
    fi = [jnp.full((_QA, 128), big_i, jnp.float32) for _ in range(nlv)]
    for t in range(_N // 128):
        v = d[:, t * 128:(t + 1) * 128]
        ix = lane_f + np.float32(t * 128)
        for lv in range(nlv):
            lt = (v < fv[lv]) | ((v == fv[lv]) & (ix < fi[lv]))
            nv = jnp.maximum(v, fv[lv])
            ni = jnp.where(lt, fi[lv], ix)
            fv[lv] = jnp.minimum(v, fv[lv])
            fi[lv] = jnp.where(lt, ix, fi[lv])
            v, ix = nv, ni

    cnt = jnp.zeros((_QA, 128), jnp.int32)
    fcols = []
    for _ in range(_K):
        m = jnp.min(fv[0], axis=1, keepdims=True)
        c = jnp.where(fv[0] == m, fi[0], big_i)
        jf = jnp.min(c, axis=1, keepdims=True)
        mask = c == jf
        cnt = cnt + mask.astype(jnp.int32)
        for lv in range(nlv - 1):
            fv[lv] = jnp.where(mask, fv[lv + 1], fv[lv])
            fi[lv] = jnp.where(mask, fi[lv + 1], fi[lv])
        fv[nlv - 1] = jnp.where(mask, big, fv[nlv - 1])
        fi[nlv - 1] = jnp.where(mask, big_i, fi[nlv - 1])
        fcols.append(jf)
    fast_idx = jnp.concatenate(fcols, axis=1).astype(jnp.int32)
    bad = jnp.max(cnt) >= nlv

    @pl.when(jnp.logical_not(bad))
    def _():
        idx_ref[0] = fast_idx + b * _N

    @pl.when(bad)
    def _():
        dd = d
        cols = []
        for _ in range(_K):
            m = jnp.min(dd, axis=1, keepdims=True)
            c = jnp.where(dd == m, iota_f, big_i)
            idxf = jnp.min(c, axis=1, keepdims=True)
            dd = jnp.where(c == idxf, big, dd)
            cols.append(idxf)
        idx_ref[0] = (jnp.concatenate(cols, axis=1).astype(jnp.int32)
                      + b * _N)

    h = (jnp.dot(x_ref[0], fc1w_ref[...], preferred_element_type=jnp.float32)
         + fc1b_ref[...])
    p1 = jnp.dot(posq, fd1w_ref[...], preferred_element_type=jnp.float32)
    table_ref[0] = jnp.concatenate([h, p1], axis=1)


def _knn_proj(pos_pad, pos_kt, x, fc1w, fc1b, fd1w):
    grid = (_B, _N // _QA)
    return pl.pallas_call(
        _knn_proj_body,
        grid=grid,
        in_specs=[
            pl.BlockSpec((1, _QA, _PW), lambda b, i: (b, i, 0)),
            pl.BlockSpec((1, _PW, _N), lambda b, i: (b, 0, 0)),
            pl.BlockSpec((1, _QA, _D), lambda b, i: (b, i, 0)),
            pl.BlockSpec((_D, _T), lambda b, i: (0, 0)),
            pl.BlockSpec((1, _T), lambda b, i: (0, 0)),
            pl.BlockSpec((_PW, _T), lambda b, i: (0, 0)),
        ],
        out_specs=[
            pl.BlockSpec((1, _QA, _WG), lambda b, i: (b, i, 0)),
            pl.BlockSpec((1, _QA, _K), lambda b, i: (b, i, 0)),
        ],
        out_shape=[
            jax.ShapeDtypeStruct((_B, _N, _WG), jnp.float32),
            jax.ShapeDtypeStruct((_B, _N, _K), jnp.int32),
        ],
    )(pos_pad, pos_kt, x, fc1w, fc1b, fd1w)


def _sc_gather(table, idx2d):
    rows = idx2d.shape[0] * _CHUNK
    info = plsc.get_sparse_core_info()
    nw = info.num_cores * info.num_subcores
    per_w = rows // nw
    nch = per_w // _CHUNK
    mesh = plsc.VectorSubcoreMesh(core_axis_name="c", subcore_axis_name="s")

    @functools.partial(
        pl.kernel,
        mesh=mesh,
        out_type=jax.ShapeDtypeStruct((rows, _WG), jnp.float32),
        scratch_types=[
            pltpu.VMEM((nch, _CHUNK), jnp.int32),
            pltpu.VMEM((_CHUNK, _WG), jnp.float32),
            pltpu.VMEM((_CHUNK, _WG), jnp.float32),
            pltpu.SemaphoreType.DMA,
            pltpu.SemaphoreType.DMA,
        ],
    )
    def gather_k(table_hbm, idx_hbm, out_hbm, idx_v, rows0, rows1,
                 sem0, sem1):
        wid = lax.axis_index("s") * info.num_cores + lax.axis_index("c")
        base = wid * per_w
        pltpu.sync_copy(idx_hbm.at[pl.ds(wid * nch, nch)], idx_v)

        pltpu.async_copy(table_hbm.at[idx_v.at[0]], rows0, sem0)

        def body(g, carry):
            pltpu.make_async_copy(table_hbm.at[idx_v.at[g]], rows0,
                                  sem0).wait()
            pltpu.async_copy(table_hbm.at[idx_v.at[g + 1]], rows1, sem1)
            pltpu.sync_copy(rows0,
                            out_hbm.at[pl.ds(base + g * _CHUNK, _CHUNK)])
            pltpu.make_async_copy(table_hbm.at[idx_v.at[g + 1]], rows1,
                                  sem1).wait()

            @pl.when(g + 2 < nch)
            def _():
                pltpu.async_copy(table_hbm.at[idx_v.at[g + 2]], rows0, sem0)

            pltpu.sync_copy(
                rows1, out_hbm.at[pl.ds(base + (g + 1) * _CHUNK, _CHUNK)])
            return carry

        lax.fori_loop(0, nch // 2, lambda i, c: body(i * 2, c), 0)

    return gather_k(table, idx2d)


def _attn_body(tq_ref, g_ref, x_ref,
               wq_ref, fd1b_ref, wkv_ref, fd2w_ref, fd2b_ref,
               fg1w_ref, fg1b_ref, fg2w_ref, fg2b_ref, fc2w_ref, fc2b_ref,
               res_ref, attn_ref):
    g = g_ref[...]
    hg = g[:, :_T]
    p1g = g[:, _T:]
    tq = tq_ref[...]
    q = jnp.dot(tq[:, :_T], wq_ref[...], preferred_element_type=jnp.float32)
    p1qb = tq[:, _T:] + fd1b_ref[...]

    t1 = (jnp.reshape(p1qb, (_QC, 1, _T))
          - jnp.reshape(p1g, (_QC, _K, _T)))
    t1 = jnp.maximum(jnp.reshape(t1, (_QC * _K, _T)), 0.0)
    pe = (jnp.dot(t1, fd2w_ref[...], preferred_element_type=jnp.float32)
          + fd2b_ref[...])

    kv = jnp.dot(hg, wkv_ref[...], preferred_element_type=jnp.float32)
    kf = kv[:, :_T]
    vf = kv[:, _T:]

    gin = (jnp.reshape(q, (_QC, 1, _T))
           - jnp.reshape(kf, (_QC, _K, _T))
           + jnp.reshape(pe, (_QC, _K, _T)))
    gin2 = jnp.reshape(gin, (_QC * _K, _T))
    logits = (jnp.dot(jnp.maximum(
                  jnp.dot(gin2, fg1w_ref[...],
                          preferred_element_type=jnp.float32) + fg1b_ref[...],
                  0.0),
              fg2w_ref[...], preferred_element_type=jnp.float32)
              + fg2b_ref[...])

    z = jnp.reshape(logits, (_QC, _K, _T)) * np.float32(1.0 / np.sqrt(_T))
    m = jnp.max(z, axis=1, keepdims=True)
    e = jnp.exp(z - m)
    attn = e / jnp.sum(e, axis=1, keepdims=True)
    for k in range(_K):
        attn_ref[0, k] = jnp.transpose(attn[:, k, :])

    w = attn * (jnp.reshape(vf, (_QC, _K, _T)) + jnp.reshape(pe, (_QC, _K, _T)))
    r = jnp.sum(w, axis=1)
    res_ref[0] = (jnp.dot(r, fc2w_ref[...], preferred_element_type=jnp.float32)
                  + fc2b_ref[...] + x_ref[0])


def _attn(table, gath, x, wq, fd1b, wkv, fd2w, fd2b,
          fg1w, fg1b, fg2w, fg2b, fc2w, fc2b):
    grid = (_B, _N // _QC)
    nblk = _N // _QC
    wspec = lambda shape: pl.BlockSpec(shape, lambda b, i: tuple(0 for _ in shape))
    return pl.pallas_call(
        _attn_body,
        grid=grid,
        in_specs=[
            pl.BlockSpec((_QC, _WG), lambda b, i: (b * nblk + i, 0)),
            pl.BlockSpec((_QC * _K, _WG), lambda b, i: (b * nblk + i, 0)),
            pl.BlockSpec((1, _QC, _D), lambda b, i: (b, i, 0)),
            wspec((_T, _T)), wspec((1, _T)),
            wspec((_T, 2 * _T)),
            wspec((_T, _T)), wspec((1, _T)),
            wspec((_T, _T)), wspec((1, _T)), wspec((_T, _T)), wspec((1, _T)),
            wspec((_T, _D)), wspec((1, _D)),
        ],
        out_specs=[
            pl.BlockSpec((1, _QC, _D), lambda b, i: (b, i, 0)),
            pl.BlockSpec((1, _K, _T, _QC), lambda b, i: (b, 0, 0, i)),
        ],
        out_shape=[
            jax.ShapeDtypeStruct((_B, _N, _D), jnp.float32),
            jax.ShapeDtypeStruct((_B, _K, _T, _N), jnp.float32),
        ],
    )(table, gath, x, wq, fd1b, wkv, fd2w, fd2b,
      fg1w, fg1b, fg2w, fg2b, fc2w, fc2b)


def kernel(x, pos, fc1_W, fc1_b, fc2_W, fc2_b, fd1_W, fd1_b, fd2_W, fd2_b,
           fg1_W, fg1_b, fg2_W, fg2_b, wq_W, wk_W, wv_W):
    pos_pad = jnp.pad(pos, ((0, 0), (0, 0), (0, _PW - 3)))
    pos_kt = jnp.transpose(pos_pad, (0, 2, 1))
    fd1p = jnp.pad(fd1_W, ((0, _PW - 3), (0, 0)))

    table, gidx = _knn_proj(pos_pad, pos_kt, x,
                            fc1_W, fc1_b.reshape(1, _T), fd1p)

    table = table.reshape(_B * _N, _WG)
    idx2d = gidx.reshape(_B * _N * _K // _CHUNK, _CHUNK)
    gath = _sc_gather(table, idx2d)

    res, attn_t = _attn(table, gath, x, wq_W, fd1_b.reshape(1, _T),
                        jnp.concatenate([wk_W, wv_W], axis=1),
                        fd2_W, fd2_b.reshape(1, _T),
                        fg1_W, fg1_b.reshape(1, _T), fg2_W, fg2_b.reshape(1, _T),
                        fc2_W, fc2_b.reshape(1, _D))
    return (res, jnp.transpose(attn_t, (0, 3, 1, 2)))
